# baseline (speedup 1.0000x reference)
import functools
import jax
import jax.numpy as jnp
from jax import lax
from jax.experimental import pallas as pl
from jax.experimental.pallas import tpu as pltpu
from jax.experimental.pallas import tpu_sc as plsc

_N = 100000
_C = 512
_NCLS = 20
_NSP = 8
_SP0 = 8
_K = 25
_BN = 2000
_NB = _N // _BN
_BP = 4000
_NP = _N // _BP
_TS = 800
_NT = _N // _TS
_NW = 32


def _logits_kernel(feat_ref, coord_ref, w_ref, b_ref,
                   logits_ref, labt_ref, part_ref):
    logits = jnp.dot(feat_ref[...], w_ref[...],
                     preferred_element_type=jnp.float32) + b_ref[...]
    logits_ref[...] = logits
    m = jnp.max(logits, axis=1, keepdims=True)
    lane = jax.lax.broadcasted_iota(jnp.int32, logits.shape, 1)
    label = jnp.min(jnp.where(logits == m, lane, _NCLS),
                    axis=1, keepdims=True)
    labt_ref[0] = jnp.transpose(label.astype(jnp.float32))
    cls = jax.lax.broadcasted_iota(jnp.int32, (_BN, _NSP), 1) + _SP0
    onehot = (label == cls).astype(jnp.float32)
    coordaug = jnp.concatenate(
        [coord_ref[...], jnp.ones((_BN, 1), jnp.float32)], axis=1)
    part_ref[0] = jax.lax.dot_general(
        coordaug, onehot, (((0,), (0,)), ((), ())),
        preferred_element_type=jnp.float32)


def _topk_kernel(coordt_ref, part_ref, labt_ref, d2sel_ref, meta_ref,
                 candv_scr, candc_scr, scr_ref):
    i = pl.program_id(0)

    @pl.when(i == 0)
    def _cents():
        s = part_ref[0]
        for j in range(1, _NB):
            s = s + part_ref[j]
        t = jnp.transpose(s)
        cnt = jnp.maximum(t[:, 3:4], 1.0)
        scr_ref[:, 3:6] = t[:, 0:3] / cnt

    cx = scr_ref[:, 3:4]
    cy = scr_ref[:, 4:5]
    cz = scr_ref[:, 5:6]
    dx = coordt_ref[0, 0:1, :] - cx
    dy = coordt_ref[0, 1:2, :] - cy
    dz = coordt_ref[0, 2:3, :] - cz
    d2 = dx * dx + dy * dy + dz * dz
    labt = labt_ref[0]
    cls = jax.lax.broadcasted_iota(
        jnp.int32, (_NSP, 1), 0).astype(jnp.float32) + float(_SP0)
    onehot = labt == cls
    d2sel_ref[0] = jnp.sum(jnp.where(onehot, d2, 0.0),
                           axis=0, keepdims=True)
    d2m = d2
    for it in range(_K):
        vmin = jnp.min(d2m, axis=1, keepdims=True)
        eq = d2m == vmin
        candv_scr[pl.ds(_NSP * i, _NSP), it:it + 1] = vmin
        candc_scr[pl.ds(_NSP * i, _NSP), it:it + 1] = jnp.sum(
            eq.astype(jnp.float32), axis=1, keepdims=True)
        d2m = jnp.where(eq, jnp.inf, d2m)

    @pl.when(i == _NP - 1)
    def _merge():
        cv = jnp.concatenate(
            [candv_scr[_NSP * j:_NSP * (j + 1), :] for j in range(_NP)],
            axis=1)
        cc = jnp.concatenate(
            [candc_scr[_NSP * j:_NSP * (j + 1), :] for j in range(_NP)],
            axis=1)
        active = jnp.ones((_NSP, 1), dtype=jnp.bool_)
        cum = jnp.zeros((_NSP, 1), dtype=jnp.float32)
        thr = jnp.zeros((_NSP, 1), dtype=jnp.float32)
        for _ in range(_K):
            vmin = jnp.min(cv, axis=1, keepdims=True)
            eqm = cv == vmin
            csum = jnp.sum(jnp.where(eqm, cc, 0.0), axis=1, keepdims=True)
            newcum = cum + csum
            hit = active & (newcum >= _K)
            thr = jnp.where(hit, vmin, thr)
            active = active & (~hit)
            cum = newcum
            cv = jnp.where(eqm, jnp.inf, cv)
        out = jnp.concatenate(
            [thr, jnp.zeros((_NSP, 7), jnp.float32)], axis=1)
        meta_ref[...] = out


def _sc_apply_body(logits_hbm, lab_hbm, d2_hbm, meta_hbm, out_hbm,
                   lvm, labvm, d2vm, metavm):
    wid = lax.axis_index("s") * 2 + lax.axis_index("c")
    pltpu.sync_copy(meta_hbm, metavm)
    ntiles = (_NT - wid + _NW - 1) // _NW

    def tile_body(t, carry):
        tile = wid + t * _NW
        tb = tile * _TS
        pltpu.sync_copy(logits_hbm.at[pl.ds(tb, _TS)], lvm)
        pltpu.sync_copy(lab_hbm.at[pl.ds(tb, _TS)], labvm)
        pltpu.sync_copy(d2_hbm.at[pl.ds(tb, _TS)], d2vm)

        def grp(k, c):
            pidx = k * 16 + lax.iota(jnp.int32, 16)
            lab = labvm[pl.ds(k * 16, 16)]
            labi = lab.astype(jnp.int32) - _SP0
            insp = (labi >= 0) & (labi < _NSP)
            si = jnp.where(insp, labi, 0)
            z16 = jnp.zeros((16,), jnp.int32)
            thr = plsc.load_gather(metavm, [si, z16])
            d2a = d2vm[pl.ds(k * 16, 16)]
            reset = insp & (d2a > thr)
            for j in range(_NCLS):
                tv = jnp.full((16,), 10.0 if j == 1 else 0.0, jnp.float32)
                plsc.store_scatter(lvm, [pidx, z16 + j], tv, mask=reset)
            return c

        lax.fori_loop(0, _TS // 16, grp, 0)
        pltpu.sync_copy(lvm, out_hbm.at[pl.ds(tb, _TS)])
        return carry

    lax.fori_loop(0, ntiles, tile_body, 0)


def _sc_apply(logits, labflat, d2, meta):
    mesh = plsc.VectorSubcoreMesh(core_axis_name="c", subcore_axis_name="s")
    fn = functools.partial(
        pl.kernel, mesh=mesh,
        out_type=jax.ShapeDtypeStruct((_N, _NCLS), jnp.float32),
        scratch_types=[
            pltpu.VMEM((_TS, _NCLS), jnp.float32),
            pltpu.VMEM((_TS,), jnp.float32),
            pltpu.VMEM((_TS,), jnp.float32),
            pltpu.VMEM((_NSP, 8), jnp.float32),
        ],
        compiler_params=pltpu.CompilerParams(needs_layout_passes=False),
    )(_sc_apply_body)
    return fn(logits, labflat, d2, meta)


def kernel(feat, coord, W, b):
    b2 = b.reshape(1, _NCLS)
    coordt = coord.reshape(_NP, _BP, 3).transpose(0, 2, 1)
    logits, labt, parts = pl.pallas_call(
        _logits_kernel,
        grid=(_NB,),
        in_specs=[
            pl.BlockSpec((_BN, _C), lambda i: (i, 0)),
            pl.BlockSpec((_BN, 3), lambda i: (i, 0)),
            pl.BlockSpec((_C, _NCLS), lambda i: (0, 0)),
            pl.BlockSpec((1, _NCLS), lambda i: (0, 0)),
        ],
        out_specs=[
            pl.BlockSpec((_BN, _NCLS), lambda i: (i, 0)),
            pl.BlockSpec((1, 1, _BN), lambda i: (i, 0, 0)),
            pl.BlockSpec((1, 4, _NSP), lambda i: (i, 0, 0)),
        ],
        out_shape=[
            jax.ShapeDtypeStruct((_N, _NCLS), jnp.float32),
            jax.ShapeDtypeStruct((_NB, 1, _BN), jnp.float32),
            jax.ShapeDtypeStruct((_NB, 4, _NSP), jnp.float32),
        ],
        compiler_params=pltpu.CompilerParams(
            dimension_semantics=("parallel",)),
    )(feat, coord, W, b2)

    labt4 = labt.reshape(_NP, 1, _BP)
    d2sel, meta = pl.pallas_call(
        _topk_kernel,
        grid=(_NP,),
        in_specs=[
            pl.BlockSpec((1, 3, _BP), lambda i: (i, 0, 0)),
            pl.BlockSpec((_NB, 4, _NSP), lambda i: (0, 0, 0)),
            pl.BlockSpec((1, 1, _BP), lambda i: (i, 0, 0)),
        ],
        out_specs=[
            pl.BlockSpec((1, 1, _BP), lambda i: (i, 0, 0)),
            pl.BlockSpec((_NSP, 8), lambda i: (0, 0)),
        ],
        out_shape=[
            jax.ShapeDtypeStruct((_NP, 1, _BP), jnp.float32),
            jax.ShapeDtypeStruct((_NSP, 8), jnp.float32),
        ],
        scratch_shapes=[
            pltpu.VMEM((_NP * _NSP, _K), jnp.float32),
            pltpu.VMEM((_NP * _NSP, _K), jnp.float32),
            pltpu.VMEM((_NSP, 8), jnp.float32),
        ],
        compiler_params=pltpu.CompilerParams(
            dimension_semantics=("arbitrary",)),
    )(coordt, parts, labt4)

    labflat = labt.reshape(_N)
    return _sc_apply(logits, labflat, d2sel.reshape(_N), meta)
